# baseline (speedup 1.0000x reference)
import functools

import jax
import jax.numpy as jnp
from jax import lax
from jax.experimental import pallas as pl
from jax.experimental.pallas import tpu as pltpu
from jax.experimental.pallas import tpu_sc as plsc

_V = 64
_B = 32
_L = 16


def _sc_body(bb_hbm, out_hbm, bbv, occ, buf, shared, sem):
    cid = lax.axis_index("c")
    sid = lax.axis_index("s")
    b = sid * 2 + cid

    pltpu.sync_copy(bb_hbm.at[b], bbv)

    zeros = jnp.zeros((_L,), jnp.float32)
    ones = jnp.ones((_L,), jnp.float32)
    for k in range(3 * _V // _L):
        occ[pl.ds(_L * k, _L)] = zeros

    iota = lax.broadcasted_iota(jnp.int32, (_L,), 0)
    bb = bbv[...]
    for d in range(3):
        mx = bb[d]
        mn = bb[d + 3]
        s = (mx - mn) * 0.015625
        for k in range(_V // _L):
            fi = (iota + _L * k).astype(jnp.float32)
            c = (fi * s + mn).astype(jnp.int32)
            msk = (c >= 0) & (c < _V)
            plsc.store_scatter(occ, [c + _V * d], ones, mask=msk)

    ox = [occ[pl.ds(_L * k, _L)] for k in range(_V // _L)]
    oy = [occ[pl.ds(_V + _L * k, _L)] for k in range(_V // _L)]
    oz = [occ[pl.ds(2 * _V + _L * k, _L)] for k in range(_V // _L)]
    for y in range(_V):
        oyv = oy[y // _L][y % _L]
        for k in range(_V // _L):
            buf[0, y, pl.ds(_L * k, _L)] = zeros
            buf[1, y, pl.ds(_L * k, _L)] = oz[k] * oyv

    sid_slot = sid
    pltpu.sync_copy(buf, shared.at[sid_slot])
    copies = []
    for x in range(_V):
        src = ox[x // _L][x % _L].astype(jnp.int32)
        copies.append(pltpu.async_copy(
            shared.at[sid_slot, src], out_hbm.at[b, x], sem))
    for cp in copies:
        cp.wait()


@functools.partial(jax.jit, static_argnames=())
def kernel(bounding_box):
    bb16 = jnp.pad(bounding_box, ((0, 0), (0, 16 - 6)))
    mesh = plsc.VectorSubcoreMesh(
        core_axis_name="c", subcore_axis_name="s", num_cores=2, num_subcores=16
    )
    out = pl.kernel(
        _sc_body,
        out_type=jax.ShapeDtypeStruct((_B, _V, _V, _V), jnp.float32),
        mesh=mesh,
        compiler_params=pltpu.CompilerParams(needs_layout_passes=False),
        scratch_types=[
            pltpu.VMEM((_L,), jnp.float32),
            pltpu.VMEM((3 * _V,), jnp.float32),
            pltpu.VMEM((2, _V, _V), jnp.float32),
            pltpu.VMEM_SHARED((16, 2, _V, _V), jnp.float32),
            pltpu.SemaphoreType.DMA,
        ],
    )(bb16)
    return out[..., None]


# speedup vs baseline: 1.3775x; 1.3775x over previous
import functools

import jax
import jax.numpy as jnp
from jax import lax
from jax.experimental import pallas as pl
from jax.experimental.pallas import tpu as pltpu
from jax.experimental.pallas import tpu_sc as plsc

_V = 64
_B = 32
_L = 16


def _sc_body(bb_hbm, out_hbm, bbv, occ, buf, sem):
    cid = lax.axis_index("c")
    sid = lax.axis_index("s")
    b = sid * 2 + cid

    pltpu.sync_copy(bb_hbm.at[b], bbv)

    zeros = jnp.zeros((_L,), jnp.float32)
    ones = jnp.ones((_L,), jnp.float32)
    for k in range(3 * _V // _L):
        occ[pl.ds(_L * k, _L)] = zeros

    iota = lax.broadcasted_iota(jnp.int32, (_L,), 0)
    bb = bbv[...]
    for d in range(3):
        mx = bb[d]
        mn = bb[d + 3]
        s = (mx - mn) * 0.015625
        for k in range(_V // _L):
            fi = (iota + _L * k).astype(jnp.float32)
            c = (fi * s + mn).astype(jnp.int32)
            msk = (c >= 0) & (c < _V)
            plsc.store_scatter(occ, [c + _V * d], ones, mask=msk)

    ox = [occ[pl.ds(_L * k, _L)] for k in range(_V // _L)]
    oy = [occ[pl.ds(_V + _L * k, _L)] for k in range(_V // _L)]
    oz = [occ[pl.ds(2 * _V + _L * k, _L)] for k in range(_V // _L)]
    for y in range(_V):
        oyv = oy[y // _L][y % _L]
        for k in range(_V // _L):
            buf[0, y, pl.ds(_L * k, _L)] = zeros
            buf[1, y, pl.ds(_L * k, _L)] = oz[k] * oyv

    copies = []
    for x in range(_V):
        src = ox[x // _L][x % _L].astype(jnp.int32)
        copies.append(pltpu.async_copy(buf.at[src], out_hbm.at[b, x], sem))
    for cp in copies:
        cp.wait()


@functools.partial(jax.jit, static_argnames=())
def kernel(bounding_box):
    bb16 = jnp.pad(bounding_box, ((0, 0), (0, 16 - 6)))
    mesh = plsc.VectorSubcoreMesh(
        core_axis_name="c", subcore_axis_name="s", num_cores=2, num_subcores=16
    )
    out = pl.kernel(
        _sc_body,
        out_type=jax.ShapeDtypeStruct((_B, _V, _V, _V), jnp.float32),
        mesh=mesh,
        compiler_params=pltpu.CompilerParams(needs_layout_passes=False),
        scratch_types=[
            pltpu.VMEM((_L,), jnp.float32),
            pltpu.VMEM((3 * _V,), jnp.float32),
            pltpu.VMEM((2, _V, _V), jnp.float32),
            pltpu.SemaphoreType.DMA,
        ],
    )(bb16)
    return out[..., None]
